# baseline (speedup 1.0000x reference)
import functools

import jax
import jax.numpy as jnp
from jax import lax
from jax.experimental import pallas as pl
from jax.experimental.pallas import tpu as pltpu
from jax.experimental.pallas import tpu_sc as plsc

L = 16
B = 4
N = 4096
NSLAB = 16
SLAB = N // NSLAB
SC_ = SLAB // L
NWB = 8
QS = N // NWB
QC = QS // L
BOFF = 3 * N
STRIDE = 3 * N + NSLAB * L
INF = 3.0e38

_MESH = plsc.VectorSubcoreMesh(core_axis_name="c", subcore_axis_name="s")


def _lanemax(v):
    m = v[0]
    for l in range(1, L):
        m = jnp.maximum(m, v[l])
    return m


def _lanemin(v):
    m = v[0]
    for l in range(1, L):
        m = jnp.minimum(m, v[l])
    return m


def _nn_pass(qv, hq, dv, hd, qbase):
    def _qchunk(qc, acc):
        qo = qbase + qc * L
        xq = qv[pl.ds(qo, L)]
        yq = qv[pl.ds(N + qo, L)]
        zq = qv[pl.ds(2 * N + qo, L)]
        nq = hq[pl.ds(qo, L)]
        aq = xq * -2.0
        bq = yq * -2.0
        cq = zq * -2.0
        xq_min = _lanemin(xq)
        xq_max = _lanemax(xq)
        yq_min = yq[0]
        yq_max = yq[L - 1]
        s = qo // SLAB

        def _chunk(j, rm):
            do = j * L
            xd = dv[pl.ds(do, L)]
            yd = dv[pl.ds(N + do, L)]
            zd = dv[pl.ds(2 * N + do, L)]
            hc = hd[pl.ds(do, L)]
            for l in range(L):
                t = hc[l] + aq * xd[l] + bq * yd[l] + cq * zd[l]
                rm = jnp.minimum(rm, t)
            return rm

        def _slabgap(t):
            tb = dv[pl.ds(BOFF + jnp.clip(t, 0, NSLAB - 1) * L, L)]
            return jnp.maximum(
                jnp.maximum(tb[0] - xq_max, xq_min - tb[1]), 0.0)

        def _ybounds(t, rem):
            base = t * SC_

            def _bs_hi(i, lohi):
                lo, hi = lohi
                mid = (lo + hi) // 2
                dyv = dv[pl.ds(N + (base + mid) * L, L)][0] - yq_max
                pred = jnp.logical_and(dyv > 0.0, dyv * dyv >= rem)
                return jnp.where(pred, lo, mid + 1), jnp.where(pred, mid, hi)

            def _bs_lo(i, lohi):
                lo, hi = lohi
                mid = (lo + hi) // 2
                dyv = yq_min - dv[pl.ds(N + (base + mid) * L, L)][L - 1]
                excl = jnp.logical_and(dyv > 0.0, dyv * dyv >= rem)
                return jnp.where(excl, mid + 1, lo), jnp.where(excl, hi, mid)

            chi, _ = lax.fori_loop(0, 4, _bs_hi,
                                   (jnp.int32(0), jnp.int32(SC_)))
            clo, _ = lax.fori_loop(0, 4, _bs_lo,
                                   (jnp.int32(0), jnp.int32(SC_)))
            ok = rem > 0.0
            clo = jnp.where(ok, clo, SC_)
            chi = jnp.where(ok, chi, 0)
            return base + clo, base + chi

        base_s = s * SC_

        def _bs_pos(i, lohi):
            lo, hi = lohi
            mid = (lo + hi) // 2
            yv = dv[pl.ds(N + (base_s + mid) * L, L)][0]
            pred = yv >= yq_min
            return jnp.where(pred, lo, mid + 1), jnp.where(pred, mid, hi)

        mpos, _ = lax.fori_loop(0, 4, _bs_pos,
                                (jnp.int32(0), jnp.int32(SC_)))
        rlo = jnp.maximum(mpos - 1, 0)
        rhi = jnp.minimum(mpos + 1, SC_ - 1)
        rm = lax.fori_loop(base_s + rlo, base_s + rhi + 1, _chunk,
                           jnp.full((L,), INF, jnp.float32))
        ub = _lanemax(rm + nq)

        gs = _slabgap(s)
        glo, ghi = _ybounds(s, ub - gs * gs)
        rm = lax.fori_loop(glo, jnp.maximum(base_s + rlo, glo), _chunk, rm)
        rm = lax.fori_loop(base_s + rhi + 1,
                           jnp.maximum(ghi, base_s + rhi + 1), _chunk, rm)
        ub = _lanemax(rm + nq)

        for dt in (-1, 1):
            t = s + dt
            gx = _slabgap(t)
            valid = jnp.logical_and(t >= 0, t < NSLAB)
            rem = jnp.where(valid, ub - gx * gx, -1.0)
            glo, ghi = _ybounds(jnp.clip(t, 0, NSLAB - 1), rem)
            rm = lax.fori_loop(glo, jnp.maximum(ghi, glo), _chunk, rm)
            ub = _lanemax(rm + nq)

        def _sb_hi(i, lohi):
            lo, hi = lohi
            mid = (lo + hi) // 2
            dx = dv[pl.ds(BOFF + mid * L, L)][0] - xq_max
            pred = jnp.logical_and(dx > 0.0, dx * dx >= ub)
            return jnp.where(pred, lo, mid + 1), jnp.where(pred, mid, hi)

        def _sb_lo(i, lohi):
            lo, hi = lohi
            mid = (lo + hi) // 2
            dx = xq_min - dv[pl.ds(BOFF + mid * L, L)][1]
            excl = jnp.logical_and(dx > 0.0, dx * dx >= ub)
            return jnp.where(excl, mid + 1, lo), jnp.where(excl, hi, mid)

        thi, _ = lax.fori_loop(0, 4, _sb_hi,
                               (jnp.int32(0), jnp.int32(NSLAB)))
        tlo, _ = lax.fori_loop(0, 4, _sb_lo,
                               (jnp.int32(0), jnp.int32(NSLAB)))

        def _far_slab(t, rm):
            gx = _slabgap(t)
            glo, ghi = _ybounds(t, ub - gx * gx)
            return lax.fori_loop(glo, jnp.maximum(ghi, glo), _chunk, rm)

        rm = lax.fori_loop(tlo, jnp.maximum(s - 1, tlo), _far_slab, rm)
        rm = lax.fori_loop(jnp.minimum(s + 2, thi), thi, _far_slab, rm)

        return acc + rm + nq

    return lax.fori_loop(0, QC, _qchunk, jnp.zeros((L,), jnp.float32))


@functools.partial(
    pl.kernel,
    out_type=jax.ShapeDtypeStruct((2 * L * NWB * B,), jnp.float32),
    mesh=_MESH,
    scratch_types=[
        pltpu.VMEM((STRIDE,), jnp.float32),
        pltpu.VMEM((STRIDE,), jnp.float32),
        pltpu.VMEM((N,), jnp.float32),
        pltpu.VMEM((N,), jnp.float32),
        pltpu.VMEM((2 * L,), jnp.float32),
    ],
)
def _cd_kernel(p1_hbm, p2_hbm, out_hbm, p1v, p2v, h1v, h2v, obuf):
    cid = lax.axis_index("c")
    sid = lax.axis_index("s")
    b = cid * 2 + sid // NWB
    k = sid % NWB
    qbase = k * QS

    pltpu.sync_copy(p1_hbm.at[pl.ds(b * STRIDE, STRIDE)], p1v)
    pltpu.sync_copy(p2_hbm.at[pl.ds(b * STRIDE, STRIDE)], p2v)

    def _norms(cv, hv):
        def _body(i, carry):
            sl = pl.ds(i * L, L)
            x = cv[pl.ds(i * L, L)]
            y = cv[pl.ds(N + i * L, L)]
            z = cv[pl.ds(2 * N + i * L, L)]
            hv[sl] = x * x + y * y + z * z
            return carry

        lax.fori_loop(0, N // L, _body, 0)

    _norms(p1v, h1v)
    _norms(p2v, h2v)

    d1vec = _nn_pass(p1v, h1v, p2v, h2v, qbase)
    d2vec = _nn_pass(p2v, h2v, p1v, h1v, qbase)

    obuf[pl.ds(0, L)] = d1vec
    obuf[pl.ds(L, L)] = d2vec
    gwid = cid * 16 + sid
    pltpu.sync_copy(obuf, out_hbm.at[pl.ds(gwid * 2 * L, 2 * L)])


def _prep(p):
    ix = jnp.argsort(p[:, :, 0], axis=1)
    ps = jnp.take_along_axis(p, ix[:, :, None], axis=1)
    xs = ps[:, :, 0]
    xlo = xs[:, ::SLAB]
    xhi = xs[:, SLAB - 1::SLAB]
    psl = ps.reshape(B, NSLAB, SLAB, 3)
    iy = jnp.argsort(psl[:, :, :, 1], axis=2)
    psl = jnp.take_along_axis(psl, iy[:, :, :, None], axis=2)
    coords = jnp.transpose(psl.reshape(B, N, 3), (0, 2, 1)).reshape(B, 3 * N)
    bounds = jnp.zeros((B, NSLAB, L), jnp.float32)
    bounds = bounds.at[:, :, 0].set(xlo).at[:, :, 1].set(xhi)
    packed = jnp.concatenate([coords, bounds.reshape(B, NSLAB * L)], axis=1)
    return packed.reshape(B * STRIDE)


def kernel(p1, p2):
    out = _cd_kernel(_prep(p1), _prep(p2))
    return jnp.sum(out) * (1.0 / (B * N))


# speedup vs baseline: 1.2583x; 1.2583x over previous
import functools

import jax
import jax.numpy as jnp
from jax import lax
from jax.experimental import pallas as pl
from jax.experimental.pallas import tpu as pltpu
from jax.experimental.pallas import tpu_sc as plsc

L = 16
B = 4
N = 4096
NSLAB = 16
SLAB = N // NSLAB
SC_ = SLAB // L
NWB = 8
QS = N // NWB
QC = QS // L
HOFF = 3 * N
BOFF = 4 * N
STRIDE = 4 * N + NSLAB * L
INF = 3.0e38

_MESH = plsc.VectorSubcoreMesh(core_axis_name="c", subcore_axis_name="s")


def _treemax(v):
    t = [v[l] for l in range(L)]
    while len(t) > 1:
        t = [jnp.maximum(t[i], t[i + 1]) for i in range(0, len(t), 2)]
    return t[0]


def _treemin(v):
    t = [v[l] for l in range(L)]
    while len(t) > 1:
        t = [jnp.minimum(t[i], t[i + 1]) for i in range(0, len(t), 2)]
    return t[0]


def _nn_pass(qv, dv, k):
    def _qchunk(qc, acc):
        qo = (qc * NWB + k) * L
        xq = qv[pl.ds(qo, L)]
        yq = qv[pl.ds(N + qo, L)]
        zq = qv[pl.ds(2 * N + qo, L)]
        nq = qv[pl.ds(HOFF + qo, L)]
        aq = xq * -2.0
        bq = yq * -2.0
        cq = zq * -2.0
        xq_min = _treemin(xq)
        xq_max = _treemax(xq)
        yq_min = yq[0]
        yq_max = yq[L - 1]
        s = qo // SLAB
        base_s = s * SC_

        def _chunk(j, rm):
            do = j * L
            xd = dv[pl.ds(do, L)]
            yd = dv[pl.ds(N + do, L)]
            zd = dv[pl.ds(2 * N + do, L)]
            hc = dv[pl.ds(HOFF + do, L)]
            t = [hc[l] + aq * xd[l] + bq * yd[l] + cq * zd[l]
                 for l in range(L)]
            while len(t) > 1:
                t = [jnp.minimum(t[i], t[i + 1]) for i in range(0, len(t), 2)]
            return jnp.minimum(rm, t[0])

        @plsc.parallel_loop(base_s, base_s + SC_, unroll=4,
                            carry=jnp.full((L,), INF, jnp.float32))
        def rm(j, r):
            return _chunk(j, r)

        ub = _treemax(rm + nq)

        def _slabgap(t):
            tb = dv[pl.ds(BOFF + jnp.clip(t, 0, NSLAB - 1) * L, L)]
            return jnp.maximum(
                jnp.maximum(tb[0] - xq_max, xq_min - tb[1]), 0.0)

        def _ybounds(t, rem):
            base = t * SC_

            def _bs_hi(i, lohi):
                lo, hi = lohi
                mid = (lo + hi) // 2
                dyv = dv[pl.ds(N + (base + mid) * L, L)][0] - yq_max
                pred = jnp.logical_and(dyv > 0.0, dyv * dyv >= rem)
                return jnp.where(pred, lo, mid + 1), jnp.where(pred, mid, hi)

            def _bs_lo(i, lohi):
                lo, hi = lohi
                dyv = yq_min - dv[pl.ds(N + (base + (lo + hi) // 2) * L,
                                        L)][L - 1]
                excl = jnp.logical_and(dyv > 0.0, dyv * dyv >= rem)
                mid = (lo + hi) // 2
                return jnp.where(excl, mid + 1, lo), jnp.where(excl, hi, mid)

            chi, _ = lax.fori_loop(0, 4, _bs_hi,
                                   (jnp.int32(0), jnp.int32(SC_)))
            clo, _ = lax.fori_loop(0, 4, _bs_lo,
                                   (jnp.int32(0), jnp.int32(SC_)))
            ok = rem > 0.0
            clo = jnp.where(ok, clo, SC_)
            chi = jnp.where(ok, chi, 0)
            return base + clo, base + chi

        for dt in (-1, 1):
            t = s + dt
            gx = _slabgap(t)
            valid = jnp.logical_and(t >= 0, t < NSLAB)
            rem = jnp.where(valid, ub - gx * gx, -1.0)
            glo, ghi = _ybounds(jnp.clip(t, 0, NSLAB - 1), rem)
            rm = lax.fori_loop(glo, jnp.maximum(ghi, glo), _chunk, rm)
            ub = _treemax(rm + nq)

        def _sb_hi(i, lohi):
            lo, hi = lohi
            mid = (lo + hi) // 2
            dx = dv[pl.ds(BOFF + mid * L, L)][0] - xq_max
            pred = jnp.logical_and(dx > 0.0, dx * dx >= ub)
            return jnp.where(pred, lo, mid + 1), jnp.where(pred, mid, hi)

        def _sb_lo(i, lohi):
            lo, hi = lohi
            mid = (lo + hi) // 2
            dx = xq_min - dv[pl.ds(BOFF + mid * L, L)][1]
            excl = jnp.logical_and(dx > 0.0, dx * dx >= ub)
            return jnp.where(excl, mid + 1, lo), jnp.where(excl, hi, mid)

        thi, _ = lax.fori_loop(0, 4, _sb_hi,
                               (jnp.int32(0), jnp.int32(NSLAB)))
        tlo, _ = lax.fori_loop(0, 4, _sb_lo,
                               (jnp.int32(0), jnp.int32(NSLAB)))

        def _far_slab(t, rm):
            gx = _slabgap(t)
            glo, ghi = _ybounds(t, ub - gx * gx)
            return lax.fori_loop(glo, jnp.maximum(ghi, glo), _chunk, rm)

        rm = lax.fori_loop(tlo, jnp.maximum(s - 1, tlo), _far_slab, rm)
        rm = lax.fori_loop(jnp.minimum(s + 2, thi), thi, _far_slab, rm)

        return acc + rm + nq

    return lax.fori_loop(0, QC, _qchunk, jnp.zeros((L,), jnp.float32))


@functools.partial(
    pl.kernel,
    out_type=jax.ShapeDtypeStruct((2 * L * NWB * B,), jnp.float32),
    mesh=_MESH,
    scratch_types=[
        pltpu.VMEM((STRIDE,), jnp.float32),
        pltpu.VMEM((STRIDE,), jnp.float32),
        pltpu.VMEM((2 * L,), jnp.float32),
    ],
)
def _cd_kernel(p1_hbm, p2_hbm, out_hbm, p1v, p2v, obuf):
    cid = lax.axis_index("c")
    sid = lax.axis_index("s")
    b = cid * 2 + sid // NWB
    k = sid % NWB

    pltpu.sync_copy(p1_hbm.at[pl.ds(b * STRIDE, STRIDE)], p1v)
    pltpu.sync_copy(p2_hbm.at[pl.ds(b * STRIDE, STRIDE)], p2v)

    d1vec = _nn_pass(p1v, p2v, k)
    d2vec = _nn_pass(p2v, p1v, k)

    obuf[pl.ds(0, L)] = d1vec
    obuf[pl.ds(L, L)] = d2vec
    gwid = cid * 16 + sid
    pltpu.sync_copy(obuf, out_hbm.at[pl.ds(gwid * 2 * L, 2 * L)])


def _prep(p):
    ix = jnp.argsort(p[:, :, 0], axis=1)
    ps = jnp.take_along_axis(p, ix[:, :, None], axis=1)
    xs = ps[:, :, 0]
    xlo = xs[:, ::SLAB]
    xhi = xs[:, SLAB - 1::SLAB]
    psl = ps.reshape(B, NSLAB, SLAB, 3)
    iy = jnp.argsort(psl[:, :, :, 1], axis=2)
    psl = jnp.take_along_axis(psl, iy[:, :, :, None], axis=2)
    pp = psl.reshape(B, N, 3)
    coords = jnp.transpose(pp, (0, 2, 1)).reshape(B, 3 * N)
    norms = jnp.sum(pp * pp, axis=2)
    bounds = jnp.zeros((B, NSLAB, L), jnp.float32)
    bounds = bounds.at[:, :, 0].set(xlo).at[:, :, 1].set(xhi)
    packed = jnp.concatenate(
        [coords, norms, bounds.reshape(B, NSLAB * L)], axis=1)
    return packed.reshape(B * STRIDE)


def kernel(p1, p2):
    out = _cd_kernel(_prep(p1), _prep(p2))
    return jnp.sum(out) * (1.0 / (B * N))
